# baseline (speedup 1.0000x reference)
import functools

import jax
import jax.numpy as jnp
import numpy as np
from jax import lax
from jax.experimental import pallas as pl
from jax.experimental.pallas import tpu as pltpu, tpu_sc as plsc

NUM_CORES = 2
NUM_SUBCORES = 16
NW = NUM_CORES * NUM_SUBCORES
LANES = 16

VOCAB = 1000000
D = 64
SEQ = 20
BATCH = 4096
ROWS = BATCH * SEQ
PER_W = ROWS // NW
GROUP = 128
CHUNK = 640
GROUPS_PER_CHUNK = CHUNK // GROUP
CHUNKS = PER_W // CHUNK
IDX_ROWS = PER_W // GROUP

_MAGIC = np.int32(0x5F3759DF)


def _rsqrt(x):
    i = plsc.bitcast(x, jnp.int32)
    i = _MAGIC - lax.shift_right_logical(i, 1)
    y = plsc.bitcast(i, jnp.float32)
    neg_half_x = x * np.float32(-0.5)
    for _ in range(3):
        y = y * (neg_half_x * y * y + np.float32(1.5))
    return y


def _ln_row(buf, row, pos_v, p, gvec, bvec):
    nj = D // LANES
    x = [buf[row, pl.ds(j * LANES, LANES)] + pos_v[p, pl.ds(j * LANES, LANES)]
         for j in range(nj)]
    s = x[0] + x[1] + x[2] + x[3]
    mean = lax.broadcast(jnp.sum(s), (LANES,)) * np.float32(1.0 / D)
    d = [xj - mean for xj in x]
    sq = d[0] * d[0] + d[1] * d[1] + d[2] * d[2] + d[3] * d[3]
    var = lax.broadcast(jnp.sum(sq), (LANES,)) * np.float32(1.0 / D)
    rinv = _rsqrt(var + np.float32(1e-5))
    for j in range(nj):
        buf[row, pl.ds(j * LANES, LANES)] = d[j] * (rinv * gvec[j]) + bvec[j]


def _body(tok_ref, emb_ref, pos_ref, gam_ref, bet_ref, out_ref,
          idx_v, rows_a, rows_b, pos_v, gam_v, bet_v,
          gsem_a, gsem_b, osem):
    wid = lax.axis_index("s") * NUM_CORES + lax.axis_index("c")
    base = wid * PER_W

    pltpu.sync_copy(tok_ref.at[wid], idx_v)
    pltpu.sync_copy(pos_ref, pos_v)
    pltpu.sync_copy(gam_ref, gam_v)
    pltpu.sync_copy(bet_ref, bet_v)

    bufs = [rows_a, rows_b]
    gsems = [gsem_a, gsem_b]

    def fire_gather(c):
        buf = bufs[c % 2]
        sem = gsems[c % 2]
        return [
            pltpu.async_copy(
                emb_ref.at[idx_v.at[c * GROUPS_PER_CHUNK + g]],
                buf.at[pl.ds(g * GROUP, GROUP)],
                sem,
            )
            for g in range(GROUPS_PER_CHUNK)
        ]

    gvec = [gam_v[pl.ds(j * LANES, LANES)] for j in range(D // LANES)]
    bvec = [bet_v[pl.ds(j * LANES, LANES)] for j in range(D // LANES)]

    gd = [None] * CHUNKS
    gd[0] = fire_gather(0)
    gd[1] = fire_gather(1)

    for c in range(CHUNKS):
        buf = bufs[c % 2]
        for desc in gd[c]:
            desc.wait()

        def period(i, carry, buf=buf):
            for p in range(SEQ):
                _ln_row(buf, i * SEQ + p, pos_v, p, gvec, bvec)
            return carry

        lax.fori_loop(0, CHUNK // SEQ, period, np.int32(0))

        pltpu.sync_copy(buf, out_ref.at[pl.ds(base + c * CHUNK, CHUNK)])
        if c + 2 < CHUNKS:
            gd[c + 2] = fire_gather(c + 2)

    del osem


@functools.partial(
    pl.kernel,
    out_type=jax.ShapeDtypeStruct((ROWS, D), jnp.float32),
    mesh=plsc.VectorSubcoreMesh(
        core_axis_name="c", subcore_axis_name="s",
        num_cores=NUM_CORES, num_subcores=NUM_SUBCORES),
    scratch_types=[
        pltpu.VMEM((IDX_ROWS, GROUP), jnp.int32),
        pltpu.VMEM((CHUNK, D), jnp.float32),
        pltpu.VMEM((CHUNK, D), jnp.float32),
        pltpu.VMEM((SEQ, D), jnp.float32),
        pltpu.VMEM((D,), jnp.float32),
        pltpu.VMEM((D,), jnp.float32),
        pltpu.SemaphoreType.DMA,
        pltpu.SemaphoreType.DMA,
        pltpu.SemaphoreType.DMA,
    ],
    compiler_params=pltpu.CompilerParams(
        needs_layout_passes=False, use_tc_tiling_on_sc=False),
)
def _encoder_sc(tok_ref, emb_ref, pos_ref, gam_ref, bet_ref, out_ref,
                idx_v, rows_a, rows_b, pos_v, gam_v, bet_v,
                gsem_a, gsem_b, osem):
    _body(tok_ref, emb_ref, pos_ref, gam_ref, bet_ref, out_ref,
          idx_v, rows_a, rows_b, pos_v, gam_v, bet_v,
          gsem_a, gsem_b, osem)


@jax.jit
def kernel(tokens, embedding, pos_embedding, ln_gamma, ln_beta):
    B, L = tokens.shape
    tok = tokens.astype(jnp.int32).reshape(NW, IDX_ROWS, GROUP)
    out = _encoder_sc(tok, embedding, pos_embedding, ln_gamma, ln_beta)
    return out.reshape(B, L, D)


# speedup vs baseline: 1.3786x; 1.3786x over previous
import functools

import jax
import jax.numpy as jnp
import numpy as np
from jax import lax
from jax.experimental import pallas as pl
from jax.experimental.pallas import tpu as pltpu, tpu_sc as plsc

NUM_CORES = 2
NUM_SUBCORES = 16
NW = NUM_CORES * NUM_SUBCORES
LANES = 16

VOCAB = 1000000
D = 64
SEQ = 20
BATCH = 4096
ROWS = BATCH * SEQ
PER_W = ROWS // NW
CHUNK = 320
CHUNKS = PER_W // CHUNK

_MAGIC = np.int32(0x5F3759DF)


def _rsqrt(x):
    i = plsc.bitcast(x, jnp.int32)
    i = _MAGIC - lax.shift_right_logical(i, 1)
    y = plsc.bitcast(i, jnp.float32)
    neg_half_x = x * np.float32(-0.5)
    for _ in range(3):
        y = y * (neg_half_x * y * y + np.float32(1.5))
    return y


def _ln_row(rows_v, b, row, pos_v, p, gvec, bvec):
    nj = D // LANES
    x = [rows_v[b, row, pl.ds(j * LANES, LANES)]
         + pos_v[p, pl.ds(j * LANES, LANES)]
         for j in range(nj)]
    s = x[0] + x[1] + x[2] + x[3]
    mean = lax.broadcast(jnp.sum(s), (LANES,)) * np.float32(1.0 / D)
    d = [xj - mean for xj in x]
    sq = d[0] * d[0] + d[1] * d[1] + d[2] * d[2] + d[3] * d[3]
    var = lax.broadcast(jnp.sum(sq), (LANES,)) * np.float32(1.0 / D)
    rinv = _rsqrt(var + np.float32(1e-5))
    for j in range(nj):
        rows_v[b, row, pl.ds(j * LANES, LANES)] = (
            d[j] * (rinv * gvec[j]) + bvec[j])


def _body(tok_ref, emb_ref, pos_ref, gam_ref, bet_ref, out_ref,
          idx_v, rows_v, pos_v, gam_v, bet_v, gsem):
    wid = lax.axis_index("s") * NUM_CORES + lax.axis_index("c")
    base = wid * PER_W

    pltpu.sync_copy(tok_ref.at[wid], idx_v)
    pltpu.sync_copy(pos_ref, pos_v)
    pltpu.sync_copy(gam_ref, gam_v)
    pltpu.sync_copy(bet_ref, bet_v)

    def fire_gather(c):
        b = lax.rem(c, np.int32(2))

        def enqueue(g, carry):
            tv = idx_v[pl.ds(c * CHUNK + g * LANES, LANES)]
            for k in range(LANES):
                pltpu.async_copy(
                    emb_ref.at[tv[k]],
                    rows_v.at[b, g * LANES + k],
                    gsem.at[b],
                )
            return carry

        lax.fori_loop(0, CHUNK // LANES, enqueue, np.int32(0))

    def drain_gather(c):
        b = lax.rem(c, np.int32(2))

        def drain(r, carry):
            pltpu.make_async_copy(
                emb_ref.at[0], rows_v.at[b, r], gsem.at[b]).wait()
            return carry

        lax.fori_loop(0, CHUNK, drain, np.int32(0))

    gvec = [gam_v[pl.ds(j * LANES, LANES)] for j in range(D // LANES)]
    bvec = [bet_v[pl.ds(j * LANES, LANES)] for j in range(D // LANES)]

    fire_gather(np.int32(0))

    def chunk_body(c, carry):
        b = lax.rem(c, np.int32(2))
        drain_gather(c)

        @pl.when(c + 1 < CHUNKS)
        def _():
            fire_gather(c + 1)

        def row_body(r, carry2):
            p = lax.rem(r, np.int32(SEQ))
            _ln_row(rows_v, b, r, pos_v, p, gvec, bvec)
            return carry2

        lax.fori_loop(0, CHUNK, row_body, np.int32(0))

        pltpu.sync_copy(
            rows_v.at[b], out_ref.at[pl.ds(base + c * CHUNK, CHUNK)])
        return carry

    lax.fori_loop(0, CHUNKS, chunk_body, np.int32(0))


@functools.partial(
    pl.kernel,
    out_type=jax.ShapeDtypeStruct((ROWS, D), jnp.float32),
    mesh=plsc.VectorSubcoreMesh(
        core_axis_name="c", subcore_axis_name="s",
        num_cores=NUM_CORES, num_subcores=NUM_SUBCORES),
    scratch_types=[
        pltpu.VMEM((PER_W,), jnp.int32),
        pltpu.VMEM((2, CHUNK, D), jnp.float32),
        pltpu.VMEM((SEQ, D), jnp.float32),
        pltpu.VMEM((D,), jnp.float32),
        pltpu.VMEM((D,), jnp.float32),
        pltpu.SemaphoreType.DMA((2,)),
    ],
    compiler_params=pltpu.CompilerParams(
        needs_layout_passes=False, use_tc_tiling_on_sc=True),
)
def _encoder_sc(tok_ref, emb_ref, pos_ref, gam_ref, bet_ref, out_ref,
                idx_v, rows_v, pos_v, gam_v, bet_v, gsem):
    _body(tok_ref, emb_ref, pos_ref, gam_ref, bet_ref, out_ref,
          idx_v, rows_v, pos_v, gam_v, bet_v, gsem)


@jax.jit
def kernel(tokens, embedding, pos_embedding, ln_gamma, ln_beta):
    B, L = tokens.shape
    tok = tokens.astype(jnp.int32).reshape(NW, PER_W)
    out = _encoder_sc(tok, embedding, pos_embedding, ln_gamma, ln_beta)
    return out.reshape(B, L, D)
